# baseline (speedup 1.0000x reference)
import functools

import jax
import jax.numpy as jnp
from jax import lax
from jax.experimental import pallas as pl
from jax.experimental.pallas import tpu as pltpu
from jax.experimental.pallas import tpu_sc as plsc

N_COLORS = 1000
EMB_DIM = 50
VOCAB = 100000
BATCH = 1024

K_PAD = 64
V_TILE = 2048
V_PAD = ((VOCAB + V_TILE - 1) // V_TILE) * V_TILE
NV = V_PAD // V_TILE
B_HALF = BATCH // 2

NEG_BIG = -1e30


_SC_TILES = 32
_B_PER_TILE = BATCH // _SC_TILES
_SC_D = 128

@functools.cache
def _make_sc_gather():
    mesh = plsc.VectorSubcoreMesh(core_axis_name="c", subcore_axis_name="s")

    @functools.partial(
        pl.kernel,
        mesh=mesh,
        out_type=jax.ShapeDtypeStruct((BATCH, _SC_D), jnp.float32),
        scratch_types=[
            pltpu.VMEM((_B_PER_TILE,), jnp.int32),
            pltpu.VMEM((_B_PER_TILE, _SC_D), jnp.float32),
            pltpu.SemaphoreType.DMA,
        ],
    )
    def _sc_gather(table_hbm, idx_hbm, out_hbm, idx_v, rows_v, sem):
        wid = lax.axis_index("s") * 2 + lax.axis_index("c")
        base = wid * _B_PER_TILE
        pltpu.sync_copy(idx_hbm.at[pl.ds(base, _B_PER_TILE)], idx_v)
        pltpu.async_copy(table_hbm.at[idx_v], rows_v, sem).wait()
        pltpu.sync_copy(rows_v, out_hbm.at[pl.ds(base, _B_PER_TILE)])

    return _sc_gather


def _lse_body(e_ref, w_ref, b_ref, lse_ref, m_ref, s_ref):
    j = pl.program_id(1)

    @pl.when(j == 0)
    def _():
        m_ref[...] = jnp.full_like(m_ref, -jnp.inf)
        s_ref[...] = jnp.zeros_like(s_ref)

    e = e_ref[...]
    e = jnp.where(e >= 0, e, 0.01 * e)
    logits = lax.dot_general(
        e, w_ref[...], (((1,), (0,)), ((), ())),
        preferred_element_type=jnp.float32,
    ) + b_ref[...]
    m_old = m_ref[...]
    m_new = jnp.maximum(m_old, jnp.max(logits, axis=1, keepdims=True))
    s_ref[...] = s_ref[...] * jnp.exp(m_old - m_new) + jnp.sum(
        jnp.exp(logits - m_new), axis=1, keepdims=True)
    m_ref[...] = m_new

    @pl.when(j == pl.num_programs(1) - 1)
    def _():
        lse_ref[...] = m_ref[...] + jnp.log(s_ref[...])


def _lse_pass(e, wt, b2):
    return pl.pallas_call(
        _lse_body,
        grid=(2, NV),
        in_specs=[
            pl.BlockSpec((B_HALF, K_PAD), lambda i, j: (i, 0)),
            pl.BlockSpec((K_PAD, V_TILE), lambda i, j: (0, j)),
            pl.BlockSpec((1, V_TILE), lambda i, j: (0, j)),
        ],
        out_specs=pl.BlockSpec((B_HALF, 1), lambda i, j: (i, 0)),
        out_shape=jax.ShapeDtypeStruct((BATCH, 1), jnp.float32),
        scratch_shapes=[
            pltpu.VMEM((B_HALF, 1), jnp.float32),
            pltpu.VMEM((B_HALF, 1), jnp.float32),
        ],
        compiler_params=pltpu.CompilerParams(
            dimension_semantics=("parallel", "arbitrary")),
    )(e, wt, b2)


def _out_body(e_ref, w_ref, b_ref, lse_ref, o_ref):
    e = e_ref[...]
    e = jnp.where(e >= 0, e, 0.01 * e)
    logits = lax.dot_general(
        e, w_ref[...], (((1,), (0,)), ((), ())),
        preferred_element_type=jnp.float32,
    ) + b_ref[...]
    o_ref[...] = logits - lse_ref[...]


def _out_pass(e, wt, b2, lse):
    return pl.pallas_call(
        _out_body,
        grid=(2, NV),
        in_specs=[
            pl.BlockSpec((B_HALF, K_PAD), lambda i, j: (i, 0)),
            pl.BlockSpec((K_PAD, V_TILE), lambda i, j: (0, j)),
            pl.BlockSpec((1, V_TILE), lambda i, j: (0, j)),
            pl.BlockSpec((B_HALF, 1), lambda i, j: (i, 0)),
        ],
        out_specs=pl.BlockSpec((B_HALF, V_TILE), lambda i, j: (i, j)),
        out_shape=jax.ShapeDtypeStruct((BATCH, VOCAB), jnp.float32),
        compiler_params=pltpu.CompilerParams(
            dimension_semantics=("parallel", "arbitrary")),
    )(e, wt, b2, lse)


def kernel(x, emb_table, W, b):
    x0 = x[:, 0].astype(jnp.int32)
    table_pad = jnp.pad(emb_table, ((0, 0), (0, _SC_D - EMB_DIM)))
    wt = jnp.pad(W.T, ((0, K_PAD - EMB_DIM), (0, V_PAD - VOCAB)))
    b2 = jnp.pad(b, (0, V_PAD - VOCAB),
                 constant_values=NEG_BIG).reshape(1, V_PAD)

    e = _make_sc_gather()(table_pad, x0)[:, :K_PAD]
    lse = _lse_pass(e, wt, b2)
    out = _out_pass(e, wt, b2, lse)
    return out.reshape(BATCH, 1, VOCAB)
